# Initial kernel scaffold; baseline (speedup 1.0000x reference)
import functools

import jax
import jax.numpy as jnp
from jax import lax
from jax.experimental import pallas as pl
from jax.experimental.pallas import tpu as pltpu
from jax.experimental.pallas import tpu_sc as plsc

_N_NODES = 10000
_N_EDGES = 320000
_D = 128

_NC = 2
_NS = 16
_NW = _NC * _NS
_EPT = _N_EDGES // _NW
_K = 80
_NCHUNK = _EPT // _K
_RPT = _N_NODES // _NS


def _sc_scatter_body(zeros_hbm, eh_hbm, dst_hbm, out_hbm, idx_v, rows_v,
                     table_sh):
    c = lax.axis_index("c")
    s = lax.axis_index("s")
    base = (c * _NS + s) * _EPT
    pltpu.sync_copy(zeros_hbm, table_sh.at[pl.ds(s * _RPT, _RPT)])
    plsc.subcore_barrier()

    def chunk(i, carry):
        off = base + i * _K
        pltpu.sync_copy(dst_hbm.at[pl.ds(off, _K)], idx_v)
        pltpu.sync_copy(eh_hbm.at[pl.ds(off, _K)], rows_v)
        pltpu.sync_copy(rows_v, table_sh.at[idx_v], add=True)
        return carry

    lax.fori_loop(0, _NCHUNK, chunk, 0)
    plsc.subcore_barrier()
    pltpu.sync_copy(table_sh.at[pl.ds(s * _RPT, _RPT)],
                    out_hbm.at[pl.ds(c * _N_NODES + s * _RPT, _RPT)])


def _sc_gather_body(mail_hbm, src_hbm, out_hbm, idx_v, rows_v, sem):
    c = lax.axis_index("c")
    s = lax.axis_index("s")
    base = (c * _NS + s) * _EPT

    def chunk(i, carry):
        off = base + i * _K
        pltpu.sync_copy(src_hbm.at[pl.ds(off, _K)], idx_v)
        pltpu.async_copy(mail_hbm.at[idx_v], rows_v, sem).wait()
        pltpu.sync_copy(rows_v, out_hbm.at[pl.ds(off, _K)])
        return carry

    lax.fori_loop(0, _NCHUNK, chunk, 0)


def _tc_merge_body(parts_ref, out_ref):
    out_ref[...] = parts_ref[0] + parts_ref[1]


def _tc_final_body(g_ref, eh_ref, einit_ref, w_ref, out_ref):
    x = g_ref[...] - eh_ref[...]
    out_ref[...] = (
        jnp.dot(x, w_ref[...], preferred_element_type=jnp.float32)
        + einit_ref[...]
    )


def kernel(edge_hidden, edge_init, W, edge_index):
    src = edge_index[0]
    dst = edge_index[1]
    zeros = jnp.zeros((_RPT, _D), dtype=jnp.float32)

    mesh = plsc.VectorSubcoreMesh(core_axis_name="c", subcore_axis_name="s")

    scatter_k = pl.kernel(
        _sc_scatter_body,
        out_type=jax.ShapeDtypeStruct((_NC * _N_NODES, _D), jnp.float32),
        mesh=mesh,
        scratch_types=[
            pltpu.VMEM((_K,), jnp.int32),
            pltpu.VMEM((_K, _D), jnp.float32),
            pltpu.VMEM_SHARED((_N_NODES, _D), jnp.float32),
        ],
    )
    parts = scatter_k(zeros, edge_hidden, dst)

    mail = pl.pallas_call(
        _tc_merge_body,
        grid=(10,),
        in_specs=[pl.BlockSpec((2, _N_NODES // 10, _D), lambda i: (0, i, 0))],
        out_specs=pl.BlockSpec((_N_NODES // 10, _D), lambda i: (i, 0)),
        out_shape=jax.ShapeDtypeStruct((_N_NODES, _D), jnp.float32),
    )(parts.reshape(_NC, _N_NODES, _D))

    gather_k = pl.kernel(
        _sc_gather_body,
        out_type=jax.ShapeDtypeStruct((_N_EDGES, _D), jnp.float32),
        mesh=mesh,
        scratch_types=[
            pltpu.VMEM((_K,), jnp.int32),
            pltpu.VMEM((_K, _D), jnp.float32),
            pltpu.SemaphoreType.DMA,
        ],
    )
    g = gather_k(mail, src)

    be = 2000
    out = pl.pallas_call(
        _tc_final_body,
        grid=(_N_EDGES // be,),
        in_specs=[
            pl.BlockSpec((be, _D), lambda i: (i, 0)),
            pl.BlockSpec((be, _D), lambda i: (i, 0)),
            pl.BlockSpec((be, _D), lambda i: (i, 0)),
            pl.BlockSpec((_D, _D), lambda i: (0, 0)),
        ],
        out_specs=pl.BlockSpec((be, _D), lambda i: (i, 0)),
        out_shape=jax.ShapeDtypeStruct((_N_EDGES, _D), jnp.float32),
    )(g, edge_hidden, edge_init, W)
    return out


# speedup vs baseline: 2.3865x; 2.3865x over previous
import functools

import jax
import jax.numpy as jnp
from jax import lax
from jax.experimental import pallas as pl
from jax.experimental.pallas import tpu as pltpu
from jax.experimental.pallas import tpu_sc as plsc

_N_NODES = 10000
_N_EDGES = 320000
_D = 128

_NC = 2
_NS = 16
_NW = _NC * _NS
_EPT = _N_EDGES // _NW
_K = 80
_NCHUNK = _EPT // _K
_NPAD = 10240
_RPT = _NPAD // _NS


def _sc_scatter_body(zeros_hbm, eh_hbm, dst_hbm, out_hbm, idx_v, rows_v,
                     table_sh):
    c = lax.axis_index("c")
    s = lax.axis_index("s")
    base = (c * _NS + s) * _EPT
    pltpu.sync_copy(zeros_hbm, table_sh.at[pl.ds(s * _RPT, _RPT)])
    plsc.subcore_barrier()

    def chunk(i, carry):
        off = base + i * _K
        pltpu.sync_copy(dst_hbm.at[pl.ds(off, _K)], idx_v)
        pltpu.sync_copy(eh_hbm.at[pl.ds(off, _K)], rows_v)
        pltpu.sync_copy(rows_v, table_sh.at[idx_v], add=True)
        return carry

    lax.fori_loop(0, _NCHUNK, chunk, 0)
    plsc.subcore_barrier()
    pltpu.sync_copy(table_sh.at[pl.ds(s * _RPT, _RPT)],
                    out_hbm.at[pl.ds(c * _NPAD + s * _RPT, _RPT)])


def _sc_gather_body(mail_hbm, src_hbm, out_hbm, idx_v, rows_v, sem):
    c = lax.axis_index("c")
    s = lax.axis_index("s")
    base = (c * _NS + s) * _EPT

    def chunk(i, carry):
        off = base + i * _K
        pltpu.sync_copy(src_hbm.at[pl.ds(off, _K)], idx_v)
        pltpu.async_copy(mail_hbm.at[idx_v], rows_v, sem).wait()
        pltpu.sync_copy(rows_v, out_hbm.at[pl.ds(off, _K)])
        return carry

    lax.fori_loop(0, _NCHUNK, chunk, 0)


def _tc_merge_body(parts_ref, out_ref):
    out_ref[...] = parts_ref[0] + parts_ref[1]


def _tc_final_body(g_ref, eh_ref, einit_ref, w_ref, out_ref):
    x = g_ref[...] - eh_ref[...]
    out_ref[...] = (
        jnp.dot(x, w_ref[...], preferred_element_type=jnp.float32)
        + einit_ref[...]
    )


def kernel(edge_hidden, edge_init, W, edge_index):
    src = edge_index[0]
    dst = edge_index[1]
    zeros = jnp.zeros((_RPT, _D), dtype=jnp.float32)

    mesh = plsc.VectorSubcoreMesh(core_axis_name="c", subcore_axis_name="s")

    scatter_k = pl.kernel(
        _sc_scatter_body,
        out_type=jax.ShapeDtypeStruct((_NC * _NPAD, _D), jnp.float32),
        mesh=mesh,
        scratch_types=[
            pltpu.VMEM((_K,), jnp.int32),
            pltpu.VMEM((_K, _D), jnp.float32),
            pltpu.VMEM_SHARED((_NPAD, _D), jnp.float32),
        ],
    )
    parts = scatter_k(zeros, edge_hidden, dst)

    mail = pl.pallas_call(
        _tc_merge_body,
        grid=(10,),
        in_specs=[pl.BlockSpec((2, _NPAD // 10, _D), lambda i: (0, i, 0))],
        out_specs=pl.BlockSpec((_NPAD // 10, _D), lambda i: (i, 0)),
        out_shape=jax.ShapeDtypeStruct((_NPAD, _D), jnp.float32),
    )(parts.reshape(_NC, _NPAD, _D))

    gather_k = pl.kernel(
        _sc_gather_body,
        out_type=jax.ShapeDtypeStruct((_N_EDGES, _D), jnp.float32),
        mesh=mesh,
        scratch_types=[
            pltpu.VMEM((_K,), jnp.int32),
            pltpu.VMEM((_K, _D), jnp.float32),
            pltpu.SemaphoreType.DMA,
        ],
    )
    g = gather_k(mail, src)

    be = 2000
    out = pl.pallas_call(
        _tc_final_body,
        grid=(_N_EDGES // be,),
        in_specs=[
            pl.BlockSpec((be, _D), lambda i: (i, 0)),
            pl.BlockSpec((be, _D), lambda i: (i, 0)),
            pl.BlockSpec((be, _D), lambda i: (i, 0)),
            pl.BlockSpec((_D, _D), lambda i: (0, 0)),
        ],
        out_specs=pl.BlockSpec((be, _D), lambda i: (i, 0)),
        out_shape=jax.ShapeDtypeStruct((_N_EDGES, _D), jnp.float32),
    )(g, edge_hidden, edge_init, W)
    return out


# speedup vs baseline: 3.8906x; 1.6303x over previous
import functools

import jax
import jax.numpy as jnp
from jax import lax
from jax.experimental import pallas as pl
from jax.experimental.pallas import tpu as pltpu
from jax.experimental.pallas import tpu_sc as plsc

_N_NODES = 10000
_N_EDGES = 320000
_D = 128

_NC = 2
_NS = 16
_NW = _NC * _NS
_EPT = _N_EDGES // _NW
_K = 80
_NCHUNK = _EPT // _K
_NPAD = 10240
_RPT = _NPAD // _NS
_NBUF = 5
_PF = 3


def _sc_scatter_body(zeros_hbm, eh_hbm, dst_hbm, out_hbm, idx_v, rows_v,
                     table_sh, sem_f, sem_s):
    c = lax.axis_index("c")
    s = lax.axis_index("s")
    w = c * _NS + s
    base = w * _EPT
    pltpu.sync_copy(zeros_hbm, table_sh.at[pl.ds(s * _RPT, _RPT)])
    pltpu.sync_copy(dst_hbm.at[w], idx_v)
    plsc.subcore_barrier()

    def fire_fetch(i, b):
        pltpu.async_copy(eh_hbm.at[pl.ds(base + i * _K, _K)], rows_v.at[b],
                         sem_f.at[b])

    def wait_fetch(i, b):
        pltpu.make_async_copy(eh_hbm.at[pl.ds(base + i * _K, _K)],
                              rows_v.at[b], sem_f.at[b]).wait()

    def fire_scat(i, b):
        pltpu.async_copy(rows_v.at[b], table_sh.at[idx_v.at[i]], sem_s.at[b],
                         add=True)

    def wait_scat(i, b):
        pltpu.make_async_copy(rows_v.at[b], table_sh.at[idx_v.at[i]],
                              sem_s.at[b]).wait()

    nb, pf = 3, 2
    for b in range(pf):
        fire_fetch(b, b)

    nslots = -(-_NCHUNK // nb) * nb

    def step(t, carry):
        for u in range(nb):
            i = t * nb + u
            b = u
            bp = (b + pf) % nb

            @pl.when(jnp.logical_and(i + pf < _NCHUNK, i >= nb - pf))
            def _():
                wait_scat(i - (nb - pf), bp)

            @pl.when(i + pf < _NCHUNK)
            def _():
                fire_fetch(i + pf, bp)

            @pl.when(i < _NCHUNK)
            def _():
                wait_fetch(i, b)
                fire_scat(i, b)
        return carry

    lax.fori_loop(0, nslots // nb, step, 0)
    for b in range(nb):
        i = _NCHUNK - nb + b
        wait_scat(i, i % nb)

    plsc.subcore_barrier()
    pltpu.sync_copy(table_sh.at[pl.ds(s * _RPT, _RPT)],
                    out_hbm.at[pl.ds(c * _NPAD + s * _RPT, _RPT)])


def _sc_gather_body(mail_hbm, src_hbm, out_hbm, idx_v, rows_v, sem_g, sem_o):
    c = lax.axis_index("c")
    s = lax.axis_index("s")
    w = c * _NS + s
    base = w * _EPT
    pltpu.sync_copy(src_hbm.at[w], idx_v)

    def fire_gath(i, b):
        pltpu.async_copy(mail_hbm.at[idx_v.at[i]], rows_v.at[b], sem_g.at[b])

    def wait_gath(i, b):
        pltpu.make_async_copy(mail_hbm.at[idx_v.at[i]], rows_v.at[b],
                              sem_g.at[b]).wait()

    def fire_out(i, b):
        pltpu.async_copy(rows_v.at[b], out_hbm.at[pl.ds(base + i * _K, _K)],
                         sem_o.at[b])

    def wait_out(i, b):
        pltpu.make_async_copy(rows_v.at[b],
                              out_hbm.at[pl.ds(base + i * _K, _K)],
                              sem_o.at[b]).wait()

    for b in range(_PF):
        fire_gath(b, b)

    def step(t, carry):
        for u in range(_NBUF):
            i = t * _NBUF + u
            b = u
            bp = (b + _PF) % _NBUF

            @pl.when(jnp.logical_and(i + _PF < _NCHUNK, i >= _NBUF - _PF))
            def _():
                wait_out(i - (_NBUF - _PF), bp)

            @pl.when(i + _PF < _NCHUNK)
            def _():
                fire_gath(i + _PF, bp)

            wait_gath(i, b)
            fire_out(i, b)
        return carry

    lax.fori_loop(0, _NCHUNK // _NBUF, step, 0)
    for b in range(_NBUF):
        wait_out(_NCHUNK - _NBUF + b, b)


def _tc_merge_body(parts_ref, out_ref):
    out_ref[...] = parts_ref[0] + parts_ref[1]


def _tc_final_body(g_ref, eh_ref, einit_ref, w_ref, out_ref):
    x = g_ref[...] - eh_ref[...]
    out_ref[...] = (
        jnp.dot(x, w_ref[...], preferred_element_type=jnp.float32)
        + einit_ref[...]
    )


def kernel(edge_hidden, edge_init, W, edge_index):
    src = edge_index[0].reshape(_NW, _NCHUNK, _K)
    dst = edge_index[1].reshape(_NW, _NCHUNK, _K)
    zeros = jnp.zeros((_RPT, _D), dtype=jnp.float32)

    mesh = plsc.VectorSubcoreMesh(core_axis_name="c", subcore_axis_name="s")

    scatter_k = pl.kernel(
        _sc_scatter_body,
        out_type=jax.ShapeDtypeStruct((_NC * _NPAD, _D), jnp.float32),
        mesh=mesh,
        scratch_types=[
            pltpu.VMEM((_NCHUNK, _K), jnp.int32),
            pltpu.VMEM((3, _K, _D), jnp.float32),
            pltpu.VMEM_SHARED((_NPAD, _D), jnp.float32),
            pltpu.SemaphoreType.DMA((3,)),
            pltpu.SemaphoreType.DMA((3,)),
        ],
    )
    parts = scatter_k(zeros, edge_hidden, dst)

    mail = pl.pallas_call(
        _tc_merge_body,
        grid=(10,),
        in_specs=[pl.BlockSpec((2, _NPAD // 10, _D), lambda i: (0, i, 0))],
        out_specs=pl.BlockSpec((_NPAD // 10, _D), lambda i: (i, 0)),
        out_shape=jax.ShapeDtypeStruct((_NPAD, _D), jnp.float32),
    )(parts.reshape(_NC, _NPAD, _D))

    gather_k = pl.kernel(
        _sc_gather_body,
        out_type=jax.ShapeDtypeStruct((_N_EDGES, _D), jnp.float32),
        mesh=mesh,
        scratch_types=[
            pltpu.VMEM((_NCHUNK, _K), jnp.int32),
            pltpu.VMEM((_NBUF, _K, _D), jnp.float32),
            pltpu.SemaphoreType.DMA((_NBUF,)),
            pltpu.SemaphoreType.DMA((_NBUF,)),
        ],
    )
    g = gather_k(mail, src)

    be = 2000
    out = pl.pallas_call(
        _tc_final_body,
        grid=(_N_EDGES // be,),
        in_specs=[
            pl.BlockSpec((be, _D), lambda i: (i, 0)),
            pl.BlockSpec((be, _D), lambda i: (i, 0)),
            pl.BlockSpec((be, _D), lambda i: (i, 0)),
            pl.BlockSpec((_D, _D), lambda i: (0, 0)),
        ],
        out_specs=pl.BlockSpec((be, _D), lambda i: (i, 0)),
        out_shape=jax.ShapeDtypeStruct((_N_EDGES, _D), jnp.float32),
    )(g, edge_hidden, edge_init, W)
    return out


# speedup vs baseline: 3.9127x; 1.0057x over previous
import functools

import jax
import jax.numpy as jnp
from jax import lax
from jax.experimental import pallas as pl
from jax.experimental.pallas import tpu as pltpu
from jax.experimental.pallas import tpu_sc as plsc

_N_NODES = 10000
_N_EDGES = 320000
_D = 128

_NC = 2
_NS = 16
_NW = _NC * _NS
_EPT = _N_EDGES // _NW
_K = 80
_NCHUNK = _EPT // _K
_NPAD = 10240
_RPT = _NPAD // _NS
_NBUF = 5
_PF = 3


def _sc_scatter_body(zeros_hbm, eh_hbm, dst_hbm, out_hbm, idx_v, rows_v,
                     table_sh, sem_f, sem_s):
    c = lax.axis_index("c")
    s = lax.axis_index("s")
    w = c * _NS + s
    base = w * _EPT
    pltpu.sync_copy(zeros_hbm, table_sh.at[pl.ds(s * _RPT, _RPT)])
    pltpu.sync_copy(dst_hbm.at[w], idx_v)
    plsc.subcore_barrier()

    def fire_fetch(i, b):
        pltpu.async_copy(eh_hbm.at[pl.ds(base + i * _K, _K)], rows_v.at[b],
                         sem_f.at[b])

    def wait_fetch(i, b):
        pltpu.make_async_copy(eh_hbm.at[pl.ds(base + i * _K, _K)],
                              rows_v.at[b], sem_f.at[b]).wait()

    def fire_scat(i, b):
        pltpu.async_copy(rows_v.at[b], table_sh.at[idx_v.at[i]], sem_s.at[b],
                         add=True)

    def wait_scat(i, b):
        pltpu.make_async_copy(rows_v.at[b], table_sh.at[idx_v.at[i]],
                              sem_s.at[b]).wait()

    nb, pf = 3, 2
    for b in range(pf):
        fire_fetch(b, b)

    nslots = -(-_NCHUNK // nb) * nb

    def step(t, carry):
        for u in range(nb):
            i = t * nb + u
            b = u
            bp = (b + pf) % nb

            @pl.when(jnp.logical_and(i + pf < _NCHUNK, i >= nb - pf))
            def _():
                wait_scat(i - (nb - pf), bp)

            @pl.when(i + pf < _NCHUNK)
            def _():
                fire_fetch(i + pf, bp)

            @pl.when(i < _NCHUNK)
            def _():
                wait_fetch(i, b)
                fire_scat(i, b)
        return carry

    lax.fori_loop(0, nslots // nb, step, 0)
    for b in range(nb):
        i = _NCHUNK - nb + b
        wait_scat(i, i % nb)

    plsc.subcore_barrier()
    pltpu.sync_copy(table_sh.at[pl.ds(s * _RPT, _RPT)],
                    out_hbm.at[pl.ds(c * _NPAD + s * _RPT, _RPT)])


_NSPLIT = 5
_EPS = _N_EDGES // _NSPLIT
_EPTG = _EPS // _NW
_NCHG = _EPTG // _K


def _sc_gather_body(mail_hbm, src_hbm, out_hbm, idx_v, rows_v, sem_g, sem_o):
    c = lax.axis_index("c")
    s = lax.axis_index("s")
    w = c * _NS + s
    base = w * _EPTG
    pltpu.sync_copy(src_hbm.at[w], idx_v)

    def fire_gath(i, b):
        pltpu.async_copy(mail_hbm.at[idx_v.at[i]], rows_v.at[b], sem_g.at[b])

    def wait_gath(i, b):
        pltpu.make_async_copy(mail_hbm.at[idx_v.at[i]], rows_v.at[b],
                              sem_g.at[b]).wait()

    def fire_out(i, b):
        pltpu.async_copy(rows_v.at[b], out_hbm.at[pl.ds(base + i * _K, _K)],
                         sem_o.at[b])

    def wait_out(i, b):
        pltpu.make_async_copy(rows_v.at[b],
                              out_hbm.at[pl.ds(base + i * _K, _K)],
                              sem_o.at[b]).wait()

    for b in range(_PF):
        fire_gath(b, b)

    def step(t, carry):
        for u in range(_NBUF):
            i = t * _NBUF + u
            b = u
            bp = (b + _PF) % _NBUF

            @pl.when(jnp.logical_and(i + _PF < _NCHG, i >= _NBUF - _PF))
            def _():
                wait_out(i - (_NBUF - _PF), bp)

            @pl.when(i + _PF < _NCHG)
            def _():
                fire_gath(i + _PF, bp)

            wait_gath(i, b)
            fire_out(i, b)
        return carry

    lax.fori_loop(0, _NCHG // _NBUF, step, 0)
    for b in range(_NBUF):
        wait_out(_NCHG - _NBUF + b, b)


def _tc_merge_body(parts_ref, out_ref):
    out_ref[...] = parts_ref[0] + parts_ref[1]


def _tc_final_body(g_ref, eh_ref, einit_ref, w_ref, out_ref):
    x = g_ref[...] - eh_ref[...]
    out_ref[...] = (
        jnp.dot(x, w_ref[...], preferred_element_type=jnp.float32)
        + einit_ref[...]
    )


def _tc_final_body_acc(g_ref, eh_ref, einit_ref, w_ref, prev_ref, out_ref):
    del prev_ref
    x = g_ref[...] - eh_ref[...]
    out_ref[...] = (
        jnp.dot(x, w_ref[...], preferred_element_type=jnp.float32)
        + einit_ref[...]
    )


def kernel(edge_hidden, edge_init, W, edge_index):
    src4 = edge_index[0].reshape(_NSPLIT, _NW, _NCHG, _K)
    dst = edge_index[1].reshape(_NW, _NCHUNK, _K)
    zeros = jnp.zeros((_RPT, _D), dtype=jnp.float32)

    mesh = plsc.VectorSubcoreMesh(core_axis_name="c", subcore_axis_name="s")

    scatter_k = pl.kernel(
        _sc_scatter_body,
        out_type=jax.ShapeDtypeStruct((_NC * _NPAD, _D), jnp.float32),
        mesh=mesh,
        scratch_types=[
            pltpu.VMEM((_NCHUNK, _K), jnp.int32),
            pltpu.VMEM((3, _K, _D), jnp.float32),
            pltpu.VMEM_SHARED((_NPAD, _D), jnp.float32),
            pltpu.SemaphoreType.DMA((3,)),
            pltpu.SemaphoreType.DMA((3,)),
        ],
    )
    parts = scatter_k(zeros, edge_hidden, dst)

    mail = pl.pallas_call(
        _tc_merge_body,
        grid=(10,),
        in_specs=[pl.BlockSpec((2, _NPAD // 10, _D), lambda i: (0, i, 0))],
        out_specs=pl.BlockSpec((_NPAD // 10, _D), lambda i: (i, 0)),
        out_shape=jax.ShapeDtypeStruct((_NPAD, _D), jnp.float32),
    )(parts.reshape(_NC, _NPAD, _D))

    gather_k = pl.kernel(
        _sc_gather_body,
        out_type=jax.ShapeDtypeStruct((_EPS, _D), jnp.float32),
        mesh=mesh,
        scratch_types=[
            pltpu.VMEM((_NCHG, _K), jnp.int32),
            pltpu.VMEM((_NBUF, _K, _D), jnp.float32),
            pltpu.SemaphoreType.DMA((_NBUF,)),
            pltpu.SemaphoreType.DMA((_NBUF,)),
        ],
    )
    gs = [gather_k(mail, src4[k]) for k in range(_NSPLIT)]

    be = 2000
    bps = _EPS // be
    out = None
    for k in range(_NSPLIT):
        row_spec = pl.BlockSpec((be, _D), lambda i, k=k: (k * bps + i, 0))
        in_specs = [
            pl.BlockSpec((be, _D), lambda i: (i, 0)),
            row_spec,
            row_spec,
            pl.BlockSpec((_D, _D), lambda i: (0, 0)),
        ]
        args = [gs[k], edge_hidden, edge_init, W]
        if k == 0:
            body, alias = _tc_final_body, {}
        else:
            body, alias = _tc_final_body_acc, {4: 0}
            in_specs.append(pl.BlockSpec(memory_space=pl.ANY))
            args.append(out)
        out = pl.pallas_call(
            body,
            grid=(bps,),
            in_specs=in_specs,
            out_specs=row_spec,
            out_shape=jax.ShapeDtypeStruct((_N_EDGES, _D), jnp.float32),
            input_output_aliases=alias,
        )(*args)
    return out
